# baseline (speedup 1.0000x reference)
import jax
import jax.numpy as jnp
from jax.experimental import pallas as pl
from jax.experimental.pallas import tpu as pltpu

_K = 500
_C = 768
_TS = 1024


def _body(x_ref, wte_ref, out_e_ref, out_s_ref, out_i_ref, out_r_ref, acc_ref):
    w = wte_ref[...]
    w_sq = jnp.sum(w * w, axis=1, keepdims=True)
    wn = w * jax.lax.rsqrt(jnp.maximum(w_sq, 1e-12))

    x = x_ref[0]
    x_sq = jnp.sum(x * x, axis=1, keepdims=True)
    xn = x * jax.lax.rsqrt(jnp.maximum(x_sq, 1e-12))

    sims = jnp.dot(xn, wn.T, preferred_element_type=jnp.float32,
                   precision=jax.lax.Precision.HIGHEST)
    out_s_ref[0] = sims

    m = jnp.max(sims, axis=1, keepdims=True)
    iota_k = jax.lax.broadcasted_iota(jnp.int32, sims.shape, 1)
    idx = jnp.min(jnp.where(sims == m, iota_k, _K), axis=1, keepdims=True)
    out_i_ref[0] = idx

    onehot = (iota_k == idx).astype(jnp.float32)
    sel = jnp.dot(onehot, w, preferred_element_type=jnp.float32)
    out_e_ref[0] = sel + x

    b = pl.program_id(0)
    s = pl.program_id(1)
    nb = pl.num_programs(0)
    ns = pl.num_programs(1)

    @pl.when((b == 0) & (s == 0))
    def _init():
        acc_ref[...] = jnp.zeros_like(acc_ref)

    acc_ref[...] += jnp.sum(m).reshape(1, 1)

    @pl.when((b == nb - 1) & (s == ns - 1))
    def _flush():
        out_r_ref[...] = acc_ref[...]


def kernel(x_embed, wte):
    B, S, C = x_embed.shape
    grid = (B, S // _TS)

    out_e, out_s, out_i, out_r = pl.pallas_call(
        _body,
        grid=grid,
        in_specs=[
            pl.BlockSpec((1, _TS, C), lambda b, s: (b, s, 0)),
            pl.BlockSpec((_K, C), lambda b, s: (0, 0)),
        ],
        out_specs=[
            pl.BlockSpec((1, _TS, C), lambda b, s: (b, s, 0)),
            pl.BlockSpec((1, _TS, _K), lambda b, s: (b, s, 0)),
            pl.BlockSpec((1, _TS, 1), lambda b, s: (b, s, 0)),
            pl.BlockSpec((1, 1), lambda b, s: (0, 0)),
        ],
        out_shape=[
            jax.ShapeDtypeStruct((B, S, C), jnp.float32),
            jax.ShapeDtypeStruct((B, S, _K), jnp.float32),
            jax.ShapeDtypeStruct((B, S, 1), jnp.int32),
            jax.ShapeDtypeStruct((1, 1), jnp.float32),
        ],
        scratch_shapes=[pltpu.VMEM((1, 1), jnp.float32)],
    )(x_embed, wte)

    reduce_sim = out_r[0, 0] / jnp.float32(B)
    return out_e, reduce_sim, out_s, out_i


# speedup vs baseline: 1.8153x; 1.8153x over previous
import jax
import jax.numpy as jnp
from jax.experimental import pallas as pl
from jax.experimental.pallas import tpu as pltpu

_K = 500
_C = 768
_TS = 2048


def _body(x_ref, wte_ref, out_e_ref, out_s_ref, out_i_ref, out_r_ref, acc_ref):
    w = wte_ref[...]
    w_sq = jnp.sum(w * w, axis=1, keepdims=True)
    wn = w * jax.lax.rsqrt(jnp.maximum(w_sq, 1e-12))

    msum = jnp.zeros((1, 1), jnp.float32)
    half = _TS // 2
    for h in range(2):
        x = x_ref[0, pl.ds(h * half, half)]
        x_sq = jnp.sum(x * x, axis=1, keepdims=True)
        xn = x * jax.lax.rsqrt(jnp.maximum(x_sq, 1e-12))

        sims = jnp.dot(xn, wn.T, preferred_element_type=jnp.float32)
        out_s_ref[0, pl.ds(h * half, half)] = sims

        m = jnp.max(sims, axis=1, keepdims=True)
        iota_k = jax.lax.broadcasted_iota(jnp.int32, sims.shape, 1)
        idx = jnp.min(jnp.where(sims == m, iota_k, _K), axis=1, keepdims=True)
        out_i_ref[0, pl.ds(h * half, half)] = idx

        onehot = (iota_k == idx).astype(jnp.float32)
        sel = jnp.dot(onehot, w, preferred_element_type=jnp.float32)
        out_e_ref[0, pl.ds(h * half, half)] = sel + x
        msum = msum + jnp.sum(m).reshape(1, 1)

    b = pl.program_id(0)
    s = pl.program_id(1)
    nb = pl.num_programs(0)
    ns = pl.num_programs(1)

    @pl.when((b == 0) & (s == 0))
    def _init():
        acc_ref[...] = jnp.zeros_like(acc_ref)

    acc_ref[...] += msum

    @pl.when((b == nb - 1) & (s == ns - 1))
    def _flush():
        out_r_ref[...] = acc_ref[...]


def kernel(x_embed, wte):
    B, S, C = x_embed.shape
    grid = (B, S // _TS)

    out_e, out_s, out_i, out_r = pl.pallas_call(
        _body,
        grid=grid,
        in_specs=[
            pl.BlockSpec((1, _TS, C), lambda b, s: (b, s, 0)),
            pl.BlockSpec((_K, C), lambda b, s: (0, 0)),
        ],
        out_specs=[
            pl.BlockSpec((1, _TS, C), lambda b, s: (b, s, 0)),
            pl.BlockSpec((1, _TS, _K), lambda b, s: (b, s, 0)),
            pl.BlockSpec((1, _TS, 1), lambda b, s: (b, s, 0)),
            pl.BlockSpec((1, 1), lambda b, s: (0, 0)),
        ],
        out_shape=[
            jax.ShapeDtypeStruct((B, S, C), jnp.float32),
            jax.ShapeDtypeStruct((B, S, _K), jnp.float32),
            jax.ShapeDtypeStruct((B, S, 1), jnp.int32),
            jax.ShapeDtypeStruct((1, 1), jnp.float32),
        ],
        scratch_shapes=[pltpu.VMEM((1, 1), jnp.float32)],
    )(x_embed, wte)

    reduce_sim = out_r[0, 0] / jnp.float32(B)
    return out_e, reduce_sim, out_s, out_i
